# baseline (speedup 1.0000x reference)
import functools

import jax
import jax.numpy as jnp
from jax import lax
from jax.experimental import pallas as pl
from jax.experimental.pallas import tpu as pltpu
from jax.experimental.pallas import tpu_sc as plsc

B = 16384
MUT_SLOTS = 20
CH = 128
NW = 32
CPW = (B // NW) // CH
RPW = B // NW
L = 16
GRP = RPW // L

MAP_DIM = 32
CMD_DIM = 48
MUT_DIM = 48
AI_DIM = 16
NUM_MUT = 1000
NUM_AI = 1000

MAP_PROWS = B * MAP_DIM // 128
CMD_PROWS = B * CMD_DIM // 128
MUT_PROWS = B * MUT_DIM // 128
AI_PROWS = B * AI_DIM // 128
MAP_WROWS = MAP_PROWS // NW
CMD_WROWS = CMD_PROWS // NW
MUT_WROWS = MUT_PROWS // NW
AI_WROWS = AI_PROWS // NW


def _sc_mut_ai(mut_r, ai_r, mutation_table, ai_table):
    mesh = plsc.VectorSubcoreMesh(core_axis_name="c", subcore_axis_name="s")
    f32 = jnp.float32
    i32 = jnp.int32

    @functools.partial(
        pl.kernel,
        out_type=(
            jax.ShapeDtypeStruct((B, MUT_DIM), f32),
            jax.ShapeDtypeStruct((B, AI_DIM), f32),
        ),
        mesh=mesh,
        compiler_params=pltpu.CompilerParams(use_tc_tiling_on_sc=False),
        scratch_types=[
            pltpu.VMEM((CPW, CH), i32),
            pltpu.VMEM((MUT_SLOTS, CPW, CH), i32),
            pltpu.VMEM((RPW, AI_DIM), f32),
            pltpu.VMEM((RPW, MUT_DIM), f32),
            pltpu.SemaphoreType.DMA,
            pltpu.SemaphoreType.DMA,
        ],
    )
    def k(mut_i, ai_i, mutt, at_,
          o_mut, o_ai,
          idx_ai, idx_mut, r_ai, acc, sem_g, sem_m):
        wid = lax.axis_index("s") * 2 + lax.axis_index("c")
        cbase = wid * CPW
        rbase = wid * RPW

        c_in = [pltpu.async_copy(ai_i.at[pl.ds(cbase, CPW)], idx_ai, sem_g)]
        c_in += [
            pltpu.async_copy(mut_i.at[s, pl.ds(cbase, CPW)], idx_mut.at[s], sem_g)
            for s in range(MUT_SLOTS)
        ]
        for cp in c_in:
            cp.wait()

        cps = []
        for j in range(CPW):
            d = pl.ds(j * CH, CH)
            cps.append(pltpu.async_copy(at_.at[idx_ai.at[j]], r_ai.at[d], sem_g))

        m0 = [pltpu.async_copy(mutt.at[idx_mut.at[0, j]],
                               acc.at[pl.ds(j * CH, CH)], sem_m)
              for j in range(CPW)]
        for cp in m0:
            cp.wait()

        def slot_body(s, carry):
            for j in range(CPW):
                pltpu.async_copy(mutt.at[idx_mut.at[s, j]],
                                 acc.at[pl.ds(j * CH, CH)], sem_m, add=True)
            return carry

        lax.fori_loop(1, MUT_SLOTS, slot_body, 0)

        for cp in cps:
            cp.wait()
        out_ai = pltpu.async_copy(r_ai, o_ai.at[pl.ds(rbase, RPW)], sem_g)

        for _ in range(MUT_SLOTS - 1):
            pltpu.make_async_copy(mutt.at[pl.ds(0, RPW)], acc, sem_m).wait()
        pltpu.sync_copy(acc, o_mut.at[pl.ds(rbase, RPW)])
        out_ai.wait()

    return k(mut_r, ai_r, mutation_table, ai_table)


def _sc_map_cmd(map_r, cmd_r, map_p, cmd_p):
    mesh = plsc.VectorSubcoreMesh(core_axis_name="c", subcore_axis_name="s")
    f32 = jnp.float32
    i32 = jnp.int32

    @functools.partial(
        pl.kernel,
        out_type=(
            jax.ShapeDtypeStruct((MAP_PROWS, 128), f32),
            jax.ShapeDtypeStruct((CMD_PROWS, 128), f32),
            jax.ShapeDtypeStruct((CMD_PROWS, 128), f32),
        ),
        mesh=mesh,
        compiler_params=pltpu.CompilerParams(use_tc_tiling_on_sc=True),
        scratch_types=[
            pltpu.VMEM((CPW, CH), i32),
            pltpu.VMEM((2, CPW, CH), i32),
            pltpu.VMEM((CH, 128), f32),
            pltpu.VMEM((CH, 128), f32),
            pltpu.VMEM((CH, 128), f32),
            pltpu.VMEM((MAP_WROWS, 128), f32),
            pltpu.VMEM((CMD_WROWS, 128), f32),
            pltpu.VMEM((CMD_WROWS, 128), f32),
            pltpu.SemaphoreType.DMA,
        ],
    )
    def k(map_i, cmd_i, mp, cp_,
          o_map, o_c0, o_c1,
          idx_map, idx_cmd, dm, dc0, dc1, pm, pc0, pc1, sem):
        wid = lax.axis_index("s") * 2 + lax.axis_index("c")
        cbase = wid * CPW

        g_in = [
            pltpu.async_copy(map_i.at[pl.ds(cbase, CPW)], idx_map, sem),
            pltpu.async_copy(cmd_i.at[0, pl.ds(cbase, CPW)], idx_cmd.at[0], sem),
            pltpu.async_copy(cmd_i.at[1, pl.ds(cbase, CPW)], idx_cmd.at[1], sem),
        ]
        for cp in g_in:
            cp.wait()

        for j in range(CPW):
            gs = [
                pltpu.async_copy(mp.at[idx_map.at[j]], dm, sem),
                pltpu.async_copy(cp_.at[idx_cmd.at[0, j]], dc0, sem),
                pltpu.async_copy(cp_.at[idx_cmd.at[1, j]], dc1, sem),
            ]
            for cp in gs:
                cp.wait()

            def row_body(i, carry):
                t = j * CH + i
                for c in range(MAP_DIM // 16):
                    g = t * MAP_DIM + c * 16
                    pm[g >> 7, pl.ds(g & 127, 16)] = dm[i, pl.ds(c * 16, 16)]
                for c in range(CMD_DIM // 16):
                    g = t * CMD_DIM + c * 16
                    pc0[g >> 7, pl.ds(g & 127, 16)] = dc0[i, pl.ds(c * 16, 16)]
                    pc1[g >> 7, pl.ds(g & 127, 16)] = dc1[i, pl.ds(c * 16, 16)]
                return carry

            lax.fori_loop(0, CH, row_body, 0)

        pltpu.sync_copy(pm, o_map.at[pl.ds(wid * MAP_WROWS, MAP_WROWS)])
        pltpu.sync_copy(pc0, o_c0.at[pl.ds(wid * CMD_WROWS, CMD_WROWS)])
        pltpu.sync_copy(pc1, o_c1.at[pl.ds(wid * CMD_WROWS, CMD_WROWS)])

    return k(map_r, cmd_r, map_p, cmd_p)


def _tc_pad_pack(tt, V, D):
    BN = 2944
    Vp = ((V + 127) // 128) * 128
    tt = jnp.pad(tt, ((0, 0), (0, Vp - V)))
    grid = (Vp // BN,)

    def body(t_ref, o_ref):
        xt = t_ref[...].T
        z = jnp.zeros((BN, 128 - D), jnp.float32)
        o_ref[...] = jnp.concatenate([xt, z], axis=1)

    return pl.pallas_call(
        body,
        grid=grid,
        in_specs=[pl.BlockSpec((D, BN), lambda i: (0, i))],
        out_specs=pl.BlockSpec((BN, 128), lambda i: (i, 0)),
        out_shape=jax.ShapeDtypeStruct((Vp, 128), jnp.float32),
    )(tt)


def _tc_combine(map_e, c0, c1, mut_e, ai_e, w0t, w1t, b2):
    BM = 2048
    grid = (B // BM,)

    def body(m_ref, c0_ref, c1_ref, mu_ref, a_ref, w0_ref, w1_ref, b_ref, o_ref):
        cmd = (
            jnp.dot(c0_ref[...], w0_ref[...], preferred_element_type=jnp.float32)
            + jnp.dot(c1_ref[...], w1_ref[...], preferred_element_type=jnp.float32)
            + b_ref[...]
        )
        o_ref[...] = jnp.concatenate(
            [m_ref[...], cmd, mu_ref[...] * (1.0 / MUT_SLOTS), a_ref[...]], axis=1)

    return pl.pallas_call(
        body,
        grid=grid,
        in_specs=[
            pl.BlockSpec((BM, MAP_DIM), lambda i: (i, 0)),
            pl.BlockSpec((BM, CMD_DIM), lambda i: (i, 0)),
            pl.BlockSpec((BM, CMD_DIM), lambda i: (i, 0)),
            pl.BlockSpec((BM, MUT_DIM), lambda i: (i, 0)),
            pl.BlockSpec((BM, AI_DIM), lambda i: (i, 0)),
            pl.BlockSpec((CMD_DIM, CMD_DIM), lambda i: (0, 0)),
            pl.BlockSpec((CMD_DIM, CMD_DIM), lambda i: (0, 0)),
            pl.BlockSpec((1, CMD_DIM), lambda i: (0, 0)),
        ],
        out_specs=pl.BlockSpec((BM, MAP_DIM + CMD_DIM + MUT_DIM + AI_DIM),
                               lambda i: (i, 0)),
        out_shape=jax.ShapeDtypeStruct(
            (B, MAP_DIM + CMD_DIM + MUT_DIM + AI_DIM), jnp.float32),
    )(map_e, c0, c1, mut_e, ai_e, w0t, w1t, b2)


def kernel(map_ids, commander_ids, mutation_ids, ai_ids,
           map_table, commander_table, mutation_table, ai_table,
           combine_W, combine_b):
    nch = B // CH
    map_r = map_ids.astype(jnp.int32).reshape(nch, CH)
    ai_r = ai_ids.astype(jnp.int32).reshape(nch, CH)
    cmd_r = commander_ids.astype(jnp.int32).T.reshape(2, nch, CH)
    mut_r = mutation_ids.astype(jnp.int32).T.reshape(MUT_SLOTS, nch, CH)

    mutt_flat = mutation_table.reshape(-1)
    ait_flat = ai_table.reshape(-1)
    (map_r, ai_r, cmd_r, mut_r, mutt_flat, ait_flat, map_tt,
     cmd_tt) = lax.optimization_barrier(
        (map_r, ai_r, cmd_r, mut_r, mutt_flat, ait_flat, map_table.T,
         commander_table.T))
    mutation_table = mutt_flat.reshape(1000, MUT_DIM)
    ai_table = ait_flat.reshape(1000, AI_DIM)
    map_p = _tc_pad_pack(map_tt, 100000, MAP_DIM)
    cmd_p = _tc_pad_pack(cmd_tt, 100000, CMD_DIM)

    mut_e, ai_e = _sc_mut_ai(mut_r, ai_r, mutation_table, ai_table)
    map_pk, c0_pk, c1_pk = _sc_map_cmd(map_r, cmd_r, map_p, cmd_p)

    w0t = combine_W[:, :CMD_DIM].T
    w1t = combine_W[:, CMD_DIM:].T
    b2 = combine_b.reshape(1, CMD_DIM)
    return _tc_combine(map_pk.reshape(B, MAP_DIM),
                       c0_pk.reshape(B, CMD_DIM), c1_pk.reshape(B, CMD_DIM),
                       mut_e, ai_e, w0t, w1t, b2)


# speedup vs baseline: 1.0971x; 1.0971x over previous
import functools

import jax
import jax.numpy as jnp
from jax import lax
from jax.experimental import pallas as pl
from jax.experimental.pallas import tpu as pltpu
from jax.experimental.pallas import tpu_sc as plsc

B = 16384
MUT_SLOTS = 20
CH = 128
NW = 32
CPW = (B // NW) // CH
RPW = B // NW

MAP_DIM = 32
CMD_DIM = 48
MUT_DIM = 48
AI_DIM = 16


def _sc_gather(map_r, cmd_r, mut_r, ai_r,
               map_table, commander_table, mutation_table, ai_table):
    mesh = plsc.VectorSubcoreMesh(core_axis_name="c", subcore_axis_name="s")
    f32 = jnp.float32
    i32 = jnp.int32

    @functools.partial(
        pl.kernel,
        out_type=(
            jax.ShapeDtypeStruct((B, MAP_DIM), f32),
            jax.ShapeDtypeStruct((B, CMD_DIM), f32),
            jax.ShapeDtypeStruct((B, CMD_DIM), f32),
            jax.ShapeDtypeStruct((B, MUT_DIM), f32),
            jax.ShapeDtypeStruct((B, AI_DIM), f32),
        ),
        mesh=mesh,
        compiler_params=pltpu.CompilerParams(use_tc_tiling_on_sc=False),
        scratch_types=[
            pltpu.VMEM((CPW, CH), i32),
            pltpu.VMEM((CPW, CH), i32),
            pltpu.VMEM((2, CPW, CH), i32),
            pltpu.VMEM((MUT_SLOTS, CPW, CH), i32),
            pltpu.VMEM((RPW, MAP_DIM), f32),
            pltpu.VMEM((RPW, CMD_DIM), f32),
            pltpu.VMEM((RPW, CMD_DIM), f32),
            pltpu.VMEM((RPW, AI_DIM), f32),
            pltpu.VMEM((RPW, MUT_DIM), f32),
            pltpu.SemaphoreType.DMA,
            pltpu.SemaphoreType.DMA,
            pltpu.SemaphoreType.DMA,
        ],
    )
    def k(map_i, cmd_i, mut_i, ai_i, mt, ct, mutt, at_,
          o_map, o_c0, o_c1, o_mut, o_ai,
          idx_map, idx_ai, idx_cmd, idx_mut,
          r_map, r_c0, r_c1, r_ai, acc, sem_g, sem_m, sem_o):
        wid = lax.axis_index("s") * 2 + lax.axis_index("c")
        cbase = wid * CPW
        rbase = wid * RPW

        c_in = [
            pltpu.async_copy(map_i.at[pl.ds(cbase, CPW)], idx_map, sem_g),
            pltpu.async_copy(ai_i.at[pl.ds(cbase, CPW)], idx_ai, sem_g),
            pltpu.async_copy(cmd_i.at[0, pl.ds(cbase, CPW)], idx_cmd.at[0], sem_g),
            pltpu.async_copy(cmd_i.at[1, pl.ds(cbase, CPW)], idx_cmd.at[1], sem_g),
        ]
        c_in += [
            pltpu.async_copy(mut_i.at[s, pl.ds(cbase, CPW)], idx_mut.at[s], sem_g)
            for s in range(MUT_SLOTS)
        ]
        for cp in c_in:
            cp.wait()

        cps = []
        for j in range(CPW):
            d = pl.ds(j * CH, CH)
            cps.append(pltpu.async_copy(mt.at[idx_map.at[j]], r_map.at[d], sem_g))
            cps.append(pltpu.async_copy(ct.at[idx_cmd.at[0, j]], r_c0.at[d], sem_g))
            cps.append(pltpu.async_copy(ct.at[idx_cmd.at[1, j]], r_c1.at[d], sem_g))
            cps.append(pltpu.async_copy(at_.at[idx_ai.at[j]], r_ai.at[d], sem_g))

        m0 = [pltpu.async_copy(mutt.at[idx_mut.at[0, j]],
                               acc.at[pl.ds(j * CH, CH)], sem_m)
              for j in range(CPW)]
        for cp in m0:
            cp.wait()

        def slot_body(s, carry):
            for j in range(CPW):
                pltpu.async_copy(mutt.at[idx_mut.at[s, j]],
                                 acc.at[pl.ds(j * CH, CH)], sem_m, add=True)
            return carry

        lax.fori_loop(1, MUT_SLOTS, slot_body, 0)

        for cp in cps:
            cp.wait()
        outs = [
            pltpu.async_copy(r_map, o_map.at[pl.ds(rbase, RPW)], sem_o),
            pltpu.async_copy(r_c0, o_c0.at[pl.ds(rbase, RPW)], sem_o),
            pltpu.async_copy(r_c1, o_c1.at[pl.ds(rbase, RPW)], sem_o),
            pltpu.async_copy(r_ai, o_ai.at[pl.ds(rbase, RPW)], sem_o),
        ]

        for _ in range(MUT_SLOTS - 1):
            pltpu.make_async_copy(mutt.at[pl.ds(0, RPW)], acc, sem_m).wait()
        pltpu.sync_copy(acc, o_mut.at[pl.ds(rbase, RPW)])

        for cp in outs:
            cp.wait()

    return k(map_r, cmd_r, mut_r, ai_r,
             map_table, commander_table, mutation_table, ai_table)


def _tc_combine(map_e, c0, c1, mut_sum, ai_e, w0t, w1t, b2):
    BM = 2048
    grid = (B // BM,)

    def body(m_ref, c0_ref, c1_ref, mu_ref, a_ref, w0_ref, w1_ref, b_ref, o_ref):
        cmd = (
            jnp.dot(c0_ref[...], w0_ref[...], preferred_element_type=jnp.float32)
            + jnp.dot(c1_ref[...], w1_ref[...], preferred_element_type=jnp.float32)
            + b_ref[...]
        )
        o_ref[...] = jnp.concatenate(
            [m_ref[...], cmd, mu_ref[...] * (1.0 / MUT_SLOTS), a_ref[...]],
            axis=1,
        )

    return pl.pallas_call(
        body,
        grid=grid,
        in_specs=[
            pl.BlockSpec((BM, MAP_DIM), lambda i: (i, 0)),
            pl.BlockSpec((BM, CMD_DIM), lambda i: (i, 0)),
            pl.BlockSpec((BM, CMD_DIM), lambda i: (i, 0)),
            pl.BlockSpec((BM, MUT_DIM), lambda i: (i, 0)),
            pl.BlockSpec((BM, AI_DIM), lambda i: (i, 0)),
            pl.BlockSpec((CMD_DIM, CMD_DIM), lambda i: (0, 0)),
            pl.BlockSpec((CMD_DIM, CMD_DIM), lambda i: (0, 0)),
            pl.BlockSpec((1, CMD_DIM), lambda i: (0, 0)),
        ],
        out_specs=pl.BlockSpec((BM, MAP_DIM + CMD_DIM + MUT_DIM + AI_DIM),
                               lambda i: (i, 0)),
        out_shape=jax.ShapeDtypeStruct(
            (B, MAP_DIM + CMD_DIM + MUT_DIM + AI_DIM), jnp.float32),
    )(map_e, c0, c1, mut_sum, ai_e, w0t, w1t, b2)


def kernel(map_ids, commander_ids, mutation_ids, ai_ids,
           map_table, commander_table, mutation_table, ai_table,
           combine_W, combine_b):
    nch = B // CH
    map_r = map_ids.astype(jnp.int32).reshape(nch, CH)
    ai_r = ai_ids.astype(jnp.int32).reshape(nch, CH)
    cmd_r = commander_ids.astype(jnp.int32).T.reshape(2, nch, CH)
    mut_r = mutation_ids.astype(jnp.int32).T.reshape(MUT_SLOTS, nch, CH)

    map_e, c0, c1, mut_sum, ai_e = _sc_gather(
        map_r, cmd_r, mut_r, ai_r,
        map_table, commander_table, mutation_table, ai_table)

    w0t = combine_W[:, :CMD_DIM].T
    w1t = combine_W[:, CMD_DIM:].T
    b2 = combine_b.reshape(1, CMD_DIM)
    return _tc_combine(map_e, c0, c1, mut_sum, ai_e, w0t, w1t, b2)
